# baseline (speedup 1.0000x reference)
import functools

import jax
import jax.numpy as jnp
from jax import lax
from jax.experimental import pallas as pl
from jax.experimental.pallas import tpu as pltpu
from jax.experimental.pallas import tpu_sc as plsc


def _gather_kernel(dim, nw, nslab, s, chunk, nbuf):
    mesh = plsc.VectorSubcoreMesh(core_axis_name="c", subcore_axis_name="s")
    nc = mesh.num_cores
    ngroups = nslab // nbuf
    assert ngroups * nbuf == nslab

    @functools.partial(
        pl.kernel,
        out_type=jax.ShapeDtypeStruct((nw, nslab, s, chunk, dim), jnp.float32),
        mesh=mesh,
        scratch_types=[
            pltpu.VMEM((nslab, s, chunk), jnp.int32),
            pltpu.VMEM((nbuf, s, chunk, dim), jnp.float32),
            pltpu.SemaphoreType.DMA,
            pltpu.SemaphoreType.DMA,
        ],
        compiler_params=pltpu.CompilerParams(use_tc_tiling_on_sc=False),
    )
    def k(idx_hbm, table_hbm, out_hbm, idx_v, rows_v, gsem, wsem):
        wid = lax.axis_index("s") * nc + lax.axis_index("c")
        pltpu.sync_copy(idx_hbm.at[wid], idx_v)

        def start_gather(j, b):
            for i in range(s):
                pltpu.async_copy(
                    table_hbm.at[idx_v.at[j].at[i]], rows_v.at[b].at[i], gsem
                )

        def wait_gather(b):
            pltpu.make_async_copy(
                out_hbm.at[wid].at[0], rows_v.at[b], gsem
            ).wait()

        def start_wb(j, b):
            pltpu.async_copy(rows_v.at[b], out_hbm.at[wid].at[j], wsem)

        def wait_wb(b):
            pltpu.make_async_copy(
                rows_v.at[b], out_hbm.at[wid].at[0], wsem
            ).wait()

        for b in range(nbuf):
            start_gather(b, b)
        for b in range(nbuf):
            wait_gather(b)
            start_wb(b, b)

        @pl.loop(1, ngroups)
        def grp(g):
            j0 = g * nbuf
            for b in range(nbuf):
                wait_wb(b)
                start_gather(j0 + b, b)
            for b in range(nbuf):
                wait_gather(b)
                start_wb(j0 + b, b)

        for b in range(nbuf):
            wait_wb(b)

    return k


def kernel(input_ids, table):
    b, h = input_ids.shape
    v, d = table.shape
    n = b * h
    idx = input_ids.reshape(n).astype(jnp.int32)

    nw = 32
    chunk = 128
    s = 4
    nbuf = 2
    rows_per_w = n // nw
    nslab = rows_per_w // (s * chunk)
    assert rows_per_w * nw == n and nslab * s * chunk == rows_per_w

    idx4 = idx.reshape(nw, nslab, s, chunk)
    out = _gather_kernel(d, nw, nslab, s, chunk, nbuf)(idx4, table)
    return out.reshape(b, h, d)
